# Initial kernel scaffold; baseline (speedup 1.0000x reference)
import jax
import jax.numpy as jnp
from jax.experimental import pallas as pl


def kernel(indices, table):
    raise NotImplementedError("write your pallas kernel here")


# speedup vs baseline: 1.0951x; 1.0951x over previous
import functools

import jax
import jax.numpy as jnp
from jax import lax
from jax.experimental import pallas as pl
from jax.experimental.pallas import tpu as pltpu
from jax.experimental.pallas import tpu_sc as plsc

VOCAB_SIZE = 1000000
EMB_SIZE = 32
BATCH = 16384
HIST = 50

NUM_CORES = 2
NUM_SUBCORES = 16
NUM_WORKERS = NUM_CORES * NUM_SUBCORES

TOTAL = BATCH * HIST
PER_WORKER = TOTAL // NUM_WORKERS
CHUNK = 1024


def _gather_kernel(idx_hbm, table_hbm, out_hbm, idx_v, rows_v, sem):
    wid = lax.axis_index("s") * NUM_CORES + lax.axis_index("c")
    base = wid * PER_WORKER

    @pl.loop(0, PER_WORKER, step=CHUNK)
    def _(off):
        pltpu.sync_copy(idx_hbm.at[pl.ds(base + off, CHUNK)], idx_v)
        pltpu.async_copy(table_hbm.at[idx_v], rows_v, sem).wait()
        pltpu.sync_copy(rows_v, out_hbm.at[pl.ds(base + off, CHUNK)])


@jax.jit
def _gather(idx_flat, table):
    mesh = plsc.VectorSubcoreMesh(core_axis_name="c", subcore_axis_name="s")
    run = functools.partial(
        pl.kernel,
        mesh=mesh,
        out_type=jax.ShapeDtypeStruct((TOTAL, EMB_SIZE), jnp.float32),
        scratch_types=[
            pltpu.VMEM((CHUNK,), jnp.int32),
            pltpu.VMEM((CHUNK, EMB_SIZE), jnp.float32),
            pltpu.SemaphoreType.DMA,
        ],
        compiler_params=pltpu.CompilerParams(use_tc_tiling_on_sc=False),
    )(_gather_kernel)
    return run(idx_flat, table)


def kernel(indices, table):
    idx_flat = indices.reshape((TOTAL,)).astype(jnp.int32)
    out = _gather(idx_flat, table)
    return out.reshape((BATCH, HIST, EMB_SIZE))


# speedup vs baseline: 1.1090x; 1.0127x over previous
import functools

import jax
import jax.numpy as jnp
from jax import lax
from jax.experimental import pallas as pl
from jax.experimental.pallas import tpu as pltpu
from jax.experimental.pallas import tpu_sc as plsc

VOCAB_SIZE = 1000000
EMB_SIZE = 32
BATCH = 16384
HIST = 50

NUM_CORES = 2
NUM_SUBCORES = 16
NUM_WORKERS = NUM_CORES * NUM_SUBCORES

TOTAL = BATCH * HIST
PER_WORKER = TOTAL // NUM_WORKERS
CHUNK = 1600
N_CHUNKS = PER_WORKER // CHUNK


def _gather_kernel(idx_hbm, table_hbm, out_hbm,
                   idx_v0, idx_v1, rows_v0, rows_v1,
                   gsem0, gsem1, osem0, osem1):
    wid = lax.axis_index("s") * NUM_CORES + lax.axis_index("c")
    base = wid * PER_WORKER

    idx_v = (idx_v0, idx_v1)
    rows_v = (rows_v0, rows_v1)
    gsem = (gsem0, gsem1)
    osem = (osem0, osem1)

    def start_gather(chunk_start, b):
        pltpu.sync_copy(idx_hbm.at[pl.ds(chunk_start, CHUNK)], idx_v[b])
        pltpu.async_copy(table_hbm.at[idx_v[b]], rows_v[b], gsem[b])

    def wait_gather(b):
        pltpu.make_async_copy(table_hbm.at[idx_v[b]], rows_v[b], gsem[b]).wait()

    def start_store(chunk_start, b):
        pltpu.async_copy(rows_v[b], out_hbm.at[pl.ds(chunk_start, CHUNK)],
                         osem[b])

    def wait_store(chunk_start, b):
        pltpu.make_async_copy(rows_v[b],
                              out_hbm.at[pl.ds(chunk_start, CHUNK)],
                              osem[b]).wait()

    start_gather(base, 0)
    start_gather(base + CHUNK, 1)

    @pl.loop(2, N_CHUNKS, step=2)
    def _(g):
        cur = base + g * CHUNK
        wait_gather(0)
        start_store(cur - 2 * CHUNK, 0)
        wait_store(cur - 2 * CHUNK, 0)
        start_gather(cur, 0)

        wait_gather(1)
        start_store(cur - CHUNK, 1)
        wait_store(cur - CHUNK, 1)
        start_gather(cur + CHUNK, 1)

    tail = base + (N_CHUNKS - 2) * CHUNK
    wait_gather(0)
    start_store(tail, 0)
    wait_gather(1)
    start_store(tail + CHUNK, 1)
    wait_store(tail, 0)
    wait_store(tail + CHUNK, 1)


@jax.jit
def _gather(idx_flat, table):
    mesh = plsc.VectorSubcoreMesh(core_axis_name="c", subcore_axis_name="s")
    run = functools.partial(
        pl.kernel,
        mesh=mesh,
        out_type=jax.ShapeDtypeStruct((TOTAL, EMB_SIZE), jnp.float32),
        scratch_types=[
            pltpu.VMEM((CHUNK,), jnp.int32),
            pltpu.VMEM((CHUNK,), jnp.int32),
            pltpu.VMEM((CHUNK, EMB_SIZE), jnp.float32),
            pltpu.VMEM((CHUNK, EMB_SIZE), jnp.float32),
            pltpu.SemaphoreType.DMA,
            pltpu.SemaphoreType.DMA,
            pltpu.SemaphoreType.DMA,
            pltpu.SemaphoreType.DMA,
        ],
        compiler_params=pltpu.CompilerParams(use_tc_tiling_on_sc=False),
    )(_gather_kernel)
    return run(idx_flat, table)


def kernel(indices, table):
    idx_flat = indices.reshape((TOTAL,)).astype(jnp.int32)
    out = _gather(idx_flat, table)
    return out.reshape((BATCH, HIST, EMB_SIZE))
